# baseline (speedup 1.0000x reference)
import jax
import jax.numpy as jnp
from jax import lax
from jax.experimental import pallas as pl
from jax.experimental.pallas import tpu as pltpu
from jax.experimental.pallas import tpu_sc as plsc

B, L, D = 16384, 50, 32
NC, NS = 2, 16
NW = NC * NS
V = 1000000
NT_FULL = V // 128
TAIL = V - NT_FULL * 128
GPW = NT_FULL // NW
REM = NT_FULL - GPW * NW
BAGS_PER_W = B // NW
T = 32
N_IT = BAGS_PER_W // T
IDX_CHUNK = T * L
INV_L = 1.0 / L


def _transpose_body(wt_hbm, tail_hbm, out_hbm, t0, t1, o0, o1,
                    sin0, sin1, sout0, sout1):
    wid = lax.axis_index("s") * NC + lax.axis_index("c")
    g0 = wid * GPW

    tiles = (t0, t1)
    outs = (o0, o1)
    sins = (sin0, sin1)
    souts = (sout0, sout1)

    def issue_in(g, slot):
        it = g0 + g
        pltpu.async_copy(wt_hbm.at[:, pl.ds(it * 128, 128)], tiles[slot],
                         sins[slot])

    def do_transpose(slot):
        @pl.loop(0, 128 // 16)
        def _k(k):
            i16 = lax.broadcasted_iota(jnp.int32, (16,), 0) + k * 16
            for c in range(D):
                plsc.store_scatter(outs[slot], [i16 * D + c],
                                   tiles[slot][c, pl.ds(k * 16, 16)])

    issue_in(0, 0)
    issue_in(1, 1)

    @pl.loop(0, GPW, step=2)
    def _grp(g):
        for p in range(2):
            gg = g + p
            pltpu.make_async_copy(
                wt_hbm.at[:, pl.ds((g0 + gg) * 128, 128)], tiles[p],
                sins[p]).wait()

            @pl.when(gg >= 2)
            def _drain():
                pltpu.make_async_copy(
                    outs[p], out_hbm.at[pl.ds(0, 128 * D)], souts[p]).wait()

            do_transpose(p)
            pltpu.async_copy(
                outs[p],
                out_hbm.at[pl.ds((g0 + gg) * 128 * D, 128 * D)],
                souts[p])

            @pl.when(gg + 2 < GPW)
            def _next():
                issue_in(gg + 2, p)

    for p in range(2):
        pltpu.make_async_copy(
            outs[p], out_hbm.at[pl.ds(0, 128 * D)], souts[p]).wait()

    @pl.when(wid < REM)
    def _leftover():
        it = NW * GPW + wid
        pltpu.sync_copy(wt_hbm.at[:, pl.ds(it * 128, 128)], t0)
        do_transpose(0)
        pltpu.sync_copy(o0, out_hbm.at[pl.ds(it * 128 * D, 128 * D)])

    @pl.when(wid == REM)
    def _tail():
        pltpu.sync_copy(tail_hbm, o0.at[pl.ds(0, TAIL * D)])
        pltpu.sync_copy(o0.at[pl.ds(0, TAIL * D)],
                        out_hbm.at[pl.ds(NT_FULL * 128 * D, TAIL * D)])


def _gather_body(idx_hbm, w_hbm, out_hbm, idx_v0, idx_v1, rows_v0, rows_v1,
                 out_v, sem0, sem1):
    wid = lax.axis_index("s") * NC + lax.axis_index("c")
    base_bag = wid * BAGS_PER_W
    idx_bufs = (idx_v0, idx_v1)
    rows_bufs = (rows_v0, rows_v1)
    sems = (sem0, sem1)

    def start_gather(t, slot):
        bag0 = base_bag + t * T
        pltpu.sync_copy(idx_hbm.at[pl.ds(bag0 * L, IDX_CHUNK)],
                        idx_bufs[slot])
        return pltpu.async_copy(w_hbm.at[idx_bufs[slot]], rows_bufs[slot],
                                sems[slot])

    copies = [None, None]
    copies[0] = start_gather(0, 0)
    for t in range(N_IT):
        cur = t % 2
        if t + 1 < N_IT:
            copies[(t + 1) % 2] = start_gather(t + 1, (t + 1) % 2)
        copies[cur].wait()
        rows_v = rows_bufs[cur]

        @pl.loop(0, T, unroll=2)
        def _bag(b):
            r0 = b * L
            a0 = rows_v[r0, pl.ds(0, 16)]
            b0 = rows_v[r0 + 1, pl.ds(0, 16)]
            a1 = rows_v[r0, pl.ds(16, 16)]
            b1 = rows_v[r0 + 1, pl.ds(16, 16)]
            for j in range(2, L, 2):
                a0 = a0 + rows_v[r0 + j, pl.ds(0, 16)]
                b0 = b0 + rows_v[r0 + j + 1, pl.ds(0, 16)]
                a1 = a1 + rows_v[r0 + j, pl.ds(16, 16)]
                b1 = b1 + rows_v[r0 + j + 1, pl.ds(16, 16)]
            out_v[b, pl.ds(0, 16)] = (a0 + b0) * INV_L
            out_v[b, pl.ds(16, 16)] = (a1 + b1) * INV_L

        pltpu.sync_copy(out_v, out_hbm.at[pl.ds(base_bag + t * T, T), :])


@jax.jit
def kernel(inputs, weights):
    flat_idx = inputs.reshape(-1)
    wt = weights.T
    mesh = plsc.VectorSubcoreMesh(
        core_axis_name="c", subcore_axis_name="s",
        num_cores=NC, num_subcores=NS)
    transpose_k = pl.kernel(
        _transpose_body,
        out_type=jax.ShapeDtypeStruct((V * D,), jnp.float32),
        mesh=mesh,
        scratch_types=[
            pltpu.VMEM((D, 128), jnp.float32),
            pltpu.VMEM((D, 128), jnp.float32),
            pltpu.VMEM((128 * D,), jnp.float32),
            pltpu.VMEM((128 * D,), jnp.float32),
            pltpu.SemaphoreType.DMA,
            pltpu.SemaphoreType.DMA,
            pltpu.SemaphoreType.DMA,
            pltpu.SemaphoreType.DMA,
        ],
        compiler_params=pltpu.CompilerParams(
            use_tc_tiling_on_sc=True, needs_layout_passes=False),
    )
    tail_flat = wt[:, NT_FULL * 128:].T.reshape(-1)
    rows_flat = transpose_k(wt, tail_flat)
    w_rows = rows_flat.reshape(V, D)
    gather_k = pl.kernel(
        _gather_body,
        out_type=jax.ShapeDtypeStruct((B, D), jnp.float32),
        mesh=mesh,
        scratch_types=[
            pltpu.VMEM((IDX_CHUNK,), jnp.int32),
            pltpu.VMEM((IDX_CHUNK,), jnp.int32),
            pltpu.VMEM((IDX_CHUNK, D), jnp.float32),
            pltpu.VMEM((IDX_CHUNK, D), jnp.float32),
            pltpu.VMEM((T, D), jnp.float32),
            pltpu.SemaphoreType.DMA,
            pltpu.SemaphoreType.DMA,
        ],
        compiler_params=pltpu.CompilerParams(use_tc_tiling_on_sc=False),
    )
    return gather_k(flat_idx, w_rows)


# speedup vs baseline: 1.2271x; 1.2271x over previous
import jax
import jax.numpy as jnp
from jax import lax
from jax.experimental import pallas as pl
from jax.experimental.pallas import tpu as pltpu
from jax.experimental.pallas import tpu_sc as plsc

B, L, D = 16384, 50, 32
NC, NS = 2, 16
NW = NC * NS
BAGS_PER_W = B // NW
T = 32
N_IT = BAGS_PER_W // T
IDX_CHUNK = T * L
INV_L = 1.0 / L


def _body(idx_hbm, w_hbm, out_hbm, idx_v0, idx_v1, rows_v0, rows_v1, out_v,
          sem0, sem1):
    wid = lax.axis_index("s") * NC + lax.axis_index("c")
    base_bag = wid * BAGS_PER_W
    idx_bufs = (idx_v0, idx_v1)
    rows_bufs = (rows_v0, rows_v1)
    sems = (sem0, sem1)

    NSPLIT = 8
    SPLIT = IDX_CHUNK // NSPLIT

    def start_gather(t, slot):
        bag0 = base_bag + t * T
        pltpu.sync_copy(idx_hbm.at[pl.ds(bag0 * L, IDX_CHUNK)],
                        idx_bufs[slot])
        return [
            pltpu.async_copy(
                w_hbm.at[idx_bufs[slot].at[pl.ds(s * SPLIT, SPLIT)]],
                rows_bufs[slot].at[pl.ds(s * SPLIT, SPLIT)],
                sems[slot])
            for s in range(NSPLIT)]

    copies = [None, None]
    copies[0] = start_gather(0, 0)
    for t in range(N_IT):
        cur = t % 2
        if t + 1 < N_IT:
            copies[(t + 1) % 2] = start_gather(t + 1, (t + 1) % 2)
        for c in copies[cur]:
            c.wait()
        rows_v = rows_bufs[cur]

        @pl.loop(0, T, unroll=2)
        def _bag(b):
            r0 = b * L
            a0 = rows_v[r0, pl.ds(0, 16)]
            b0 = rows_v[r0 + 1, pl.ds(0, 16)]
            a1 = rows_v[r0, pl.ds(16, 16)]
            b1 = rows_v[r0 + 1, pl.ds(16, 16)]
            for j in range(2, L, 2):
                a0 = a0 + rows_v[r0 + j, pl.ds(0, 16)]
                b0 = b0 + rows_v[r0 + j + 1, pl.ds(0, 16)]
                a1 = a1 + rows_v[r0 + j, pl.ds(16, 16)]
                b1 = b1 + rows_v[r0 + j + 1, pl.ds(16, 16)]
            out_v[b, pl.ds(0, 16)] = (a0 + b0) * INV_L
            out_v[b, pl.ds(16, 16)] = (a1 + b1) * INV_L

        pltpu.sync_copy(out_v, out_hbm.at[pl.ds(base_bag + t * T, T), :])


@jax.jit
def kernel(inputs, weights):
    flat_idx = inputs.reshape(-1)
    mesh = plsc.VectorSubcoreMesh(
        core_axis_name="c", subcore_axis_name="s",
        num_cores=NC, num_subcores=NS)
    k = pl.kernel(
        _body,
        out_type=jax.ShapeDtypeStruct((B, D), jnp.float32),
        mesh=mesh,
        scratch_types=[
            pltpu.VMEM((IDX_CHUNK,), jnp.int32),
            pltpu.VMEM((IDX_CHUNK,), jnp.int32),
            pltpu.VMEM((IDX_CHUNK, D), jnp.float32),
            pltpu.VMEM((IDX_CHUNK, D), jnp.float32),
            pltpu.VMEM((T, D), jnp.float32),
            pltpu.SemaphoreType.DMA,
            pltpu.SemaphoreType.DMA,
        ],
        compiler_params=pltpu.CompilerParams(use_tc_tiling_on_sc=False),
    )
    return k(flat_idx, weights)


# speedup vs baseline: 1.2302x; 1.0025x over previous
import jax
import jax.numpy as jnp
from jax import lax
from jax.experimental import pallas as pl
from jax.experimental.pallas import tpu as pltpu
from jax.experimental.pallas import tpu_sc as plsc

B, L, D = 16384, 50, 32
NC, NS = 2, 16
NW = NC * NS
BAGS_PER_W = B // NW
T = 32
N_IT = BAGS_PER_W // T
IDX_CHUNK = T * L
INV_L = 1.0 / L


def _body(idx_hbm, w_hbm, out_hbm, idx_v0, idx_v1, rows_v0, rows_v1, out_v,
          sem0, sem1):
    wid = lax.axis_index("s") * NC + lax.axis_index("c")
    base_bag = wid * BAGS_PER_W
    idx_bufs = (idx_v0, idx_v1)
    rows_bufs = (rows_v0, rows_v1)
    sems = (sem0, sem1)

    NSPLIT = 20
    SPLIT = IDX_CHUNK // NSPLIT

    def start_gather(t, slot):
        bag0 = base_bag + t * T
        pltpu.sync_copy(idx_hbm.at[pl.ds(bag0 * L, IDX_CHUNK)],
                        idx_bufs[slot])
        return [
            pltpu.async_copy(
                w_hbm.at[idx_bufs[slot].at[pl.ds(s * SPLIT, SPLIT)]],
                rows_bufs[slot].at[pl.ds(s * SPLIT, SPLIT)],
                sems[slot])
            for s in range(NSPLIT)]

    copies = [None, None]
    copies[0] = start_gather(0, 0)
    for t in range(N_IT):
        cur = t % 2
        if t + 1 < N_IT:
            copies[(t + 1) % 2] = start_gather(t + 1, (t + 1) % 2)
        for c in copies[cur]:
            c.wait()
        rows_v = rows_bufs[cur]

        @pl.loop(0, T, unroll=2)
        def _bag(b):
            r0 = b * L
            a0 = rows_v[r0, pl.ds(0, 16)]
            b0 = rows_v[r0 + 1, pl.ds(0, 16)]
            a1 = rows_v[r0, pl.ds(16, 16)]
            b1 = rows_v[r0 + 1, pl.ds(16, 16)]
            for j in range(2, L, 2):
                a0 = a0 + rows_v[r0 + j, pl.ds(0, 16)]
                b0 = b0 + rows_v[r0 + j + 1, pl.ds(0, 16)]
                a1 = a1 + rows_v[r0 + j, pl.ds(16, 16)]
                b1 = b1 + rows_v[r0 + j + 1, pl.ds(16, 16)]
            out_v[b, pl.ds(0, 16)] = (a0 + b0) * INV_L
            out_v[b, pl.ds(16, 16)] = (a1 + b1) * INV_L

        pltpu.sync_copy(out_v, out_hbm.at[pl.ds(base_bag + t * T, T), :])


@jax.jit
def kernel(inputs, weights):
    flat_idx = inputs.reshape(-1)
    mesh = plsc.VectorSubcoreMesh(
        core_axis_name="c", subcore_axis_name="s",
        num_cores=NC, num_subcores=NS)
    k = pl.kernel(
        _body,
        out_type=jax.ShapeDtypeStruct((B, D), jnp.float32),
        mesh=mesh,
        scratch_types=[
            pltpu.VMEM((IDX_CHUNK,), jnp.int32),
            pltpu.VMEM((IDX_CHUNK,), jnp.int32),
            pltpu.VMEM((IDX_CHUNK, D), jnp.float32),
            pltpu.VMEM((IDX_CHUNK, D), jnp.float32),
            pltpu.VMEM((T, D), jnp.float32),
            pltpu.SemaphoreType.DMA,
            pltpu.SemaphoreType.DMA,
        ],
        compiler_params=pltpu.CompilerParams(use_tc_tiling_on_sc=False),
    )
    return k(flat_idx, weights)


# speedup vs baseline: 1.2373x; 1.0058x over previous
import jax
import jax.numpy as jnp
from jax import lax
from jax.experimental import pallas as pl
from jax.experimental.pallas import tpu as pltpu
from jax.experimental.pallas import tpu_sc as plsc

B, L, D = 16384, 50, 32
NC, NS = 2, 16
NW = NC * NS
BAGS_PER_W = B // NW
T = 32
N_IT = BAGS_PER_W // T
IDX_CHUNK = T * L
IDX_ALL = BAGS_PER_W * L
INV_L = 1.0 / L


def _body(idx_hbm, w_hbm, out_hbm, idx_v, rows_v0, rows_v1, out_v0, out_v1,
          sem0, sem1, osem0, osem1):
    wid = lax.axis_index("s") * NC + lax.axis_index("c")
    base_bag = wid * BAGS_PER_W
    rows_bufs = (rows_v0, rows_v1)
    out_bufs = (out_v0, out_v1)
    sems = (sem0, sem1)
    osems = (osem0, osem1)

    NSPLIT = 8
    SPLIT = IDX_CHUNK // NSPLIT

    pltpu.sync_copy(idx_hbm.at[pl.ds(base_bag * L, IDX_ALL)], idx_v)

    def start_gather(t, slot):
        off = t * IDX_CHUNK
        return [
            pltpu.async_copy(
                w_hbm.at[idx_v.at[pl.ds(off + s * SPLIT, SPLIT)]],
                rows_bufs[slot].at[pl.ds(s * SPLIT, SPLIT)],
                sems[slot])
            for s in range(NSPLIT)]

    copies = [None, None]
    outc = [None, None]
    copies[0] = start_gather(0, 0)
    for t in range(N_IT):
        cur = t % 2
        if t + 1 < N_IT:
            copies[(t + 1) % 2] = start_gather(t + 1, (t + 1) % 2)
        for c in copies[cur]:
            c.wait()
        rows_v = rows_bufs[cur]
        out_v = out_bufs[cur]
        if outc[cur] is not None:
            outc[cur].wait()

        @pl.loop(0, T, unroll=2)
        def _bag(b):
            r0 = b * L
            a0 = rows_v[r0, pl.ds(0, 16)]
            b0 = rows_v[r0 + 1, pl.ds(0, 16)]
            a1 = rows_v[r0, pl.ds(16, 16)]
            b1 = rows_v[r0 + 1, pl.ds(16, 16)]
            for j in range(2, L, 2):
                a0 = a0 + rows_v[r0 + j, pl.ds(0, 16)]
                b0 = b0 + rows_v[r0 + j + 1, pl.ds(0, 16)]
                a1 = a1 + rows_v[r0 + j, pl.ds(16, 16)]
                b1 = b1 + rows_v[r0 + j + 1, pl.ds(16, 16)]
            out_v[b, pl.ds(0, 16)] = (a0 + b0) * INV_L
            out_v[b, pl.ds(16, 16)] = (a1 + b1) * INV_L

        outc[cur] = pltpu.async_copy(
            out_v, out_hbm.at[pl.ds(base_bag + t * T, T), :], osems[cur])
    for c in outc:
        if c is not None:
            c.wait()


@jax.jit
def kernel(inputs, weights):
    flat_idx = inputs.reshape(-1)
    mesh = plsc.VectorSubcoreMesh(
        core_axis_name="c", subcore_axis_name="s",
        num_cores=NC, num_subcores=NS)
    k = pl.kernel(
        _body,
        out_type=jax.ShapeDtypeStruct((B, D), jnp.float32),
        mesh=mesh,
        scratch_types=[
            pltpu.VMEM((IDX_ALL,), jnp.int32),
            pltpu.VMEM((IDX_CHUNK, D), jnp.float32),
            pltpu.VMEM((IDX_CHUNK, D), jnp.float32),
            pltpu.VMEM((T, D), jnp.float32),
            pltpu.VMEM((T, D), jnp.float32),
            pltpu.SemaphoreType.DMA,
            pltpu.SemaphoreType.DMA,
            pltpu.SemaphoreType.DMA,
            pltpu.SemaphoreType.DMA,
        ],
        compiler_params=pltpu.CompilerParams(use_tc_tiling_on_sc=False),
    )
    return k(flat_idx, weights)
